# baseline (speedup 1.0000x reference)
import functools

import jax
import jax.numpy as jnp
from jax import lax
from jax.experimental import pallas as pl
from jax.experimental.pallas import tpu as pltpu
from jax.experimental.pallas import tpu_sc as plsc

D = 64
DP = 128
NW = 32
NBUF = 4


@jax.jit
def _convert(embT, tail):
    V = embT.shape[1]
    n_full = V // DP
    V128 = (n_full + 1) * DP

    mesh = plsc.VectorSubcoreMesh(core_axis_name="c", subcore_axis_name="s")

    @functools.partial(
        pl.kernel,
        out_type=jax.ShapeDtypeStruct((V128, DP), jnp.float32),
        mesh=mesh,
        scratch_types=[
            pltpu.VMEM((2, D, DP), jnp.float32),
            pltpu.VMEM((2, DP, DP), jnp.float32),
            pltpu.SemaphoreType.DMA((2,)),
            pltpu.SemaphoreType.DMA((2,)),
        ],
        compiler_params=pltpu.CompilerParams(
            needs_layout_passes=False, disable_bounds_checks=True),
    )
    def k(embT_hbm, tail_hbm, o1_hbm, in_v, out_v, gsem, ssem):
        wid = lax.axis_index("s") * 2 + lax.axis_index("c")
        n_my = (n_full - wid + NW - 1) // NW

        @pl.when(wid == NW - 1)
        def _():
            pltpu.sync_copy(tail_hbm, out_v.at[0])
            pltpu.sync_copy(out_v.at[0], o1_hbm.at[pl.ds(n_full * DP, DP)])

        def gin(t, b):
            c0 = (t * NW + wid) * DP
            return pltpu.make_async_copy(
                embT_hbm.at[:, pl.ds(c0, DP)], in_v.at[b], gsem.at[b])

        def gout(t, b):
            c0 = (t * NW + wid) * DP
            return pltpu.make_async_copy(
                out_v.at[b], o1_hbm.at[pl.ds(c0, DP)], ssem.at[b])

        rows = [lax.iota(jnp.int32, 16) + 16 * c for c in range(8)]

        @pl.when(n_my > 0)
        def _():
            gin(0, 0).start()

            def body(t, carry):
                b = lax.rem(t, 2)
                gin(t, b).wait()

                @pl.when(t + 1 < n_my)
                def _():
                    gin(t + 1, 1 - b).start()

                @pl.when(t >= 2)
                def _():
                    gout(t - 2, b).wait()

                def dcol(i2, c2):
                    for u in range(2):
                        i = i2 * 2 + u
                        ivec = jnp.full((16,), 0, jnp.int32) + i
                        for dg in range(4):
                            v = plsc.load_gather(
                                in_v.at[b], [rows[dg], ivec])
                            out_v[b, i, pl.ds(16 * dg, 16)] = v
                    return c2

                lax.fori_loop(0, DP // 2, dcol, 0)
                gout(t, b).start()
                return carry

            lax.fori_loop(0, n_my, body, 0)

            @pl.when(n_my >= 2)
            def _():
                gout(n_my - 2, lax.rem(n_my - 2, 2)).wait()
            gout(n_my - 1, lax.rem(n_my - 1, 2)).wait()

    return k(embT, tail)


@jax.jit
def _lookup(idx1d, tpad):
    B = idx1d.shape[0]
    b_per_w = B // NW
    n_b = 1024 // NW
    CH = 200
    n_grp = n_b // NBUF
    assert b_per_w == n_b * CH and n_b % NBUF == 0

    mesh = plsc.VectorSubcoreMesh(core_axis_name="c", subcore_axis_name="s")

    @functools.partial(
        pl.kernel,
        out_type=jax.ShapeDtypeStruct((1024, 200, DP), jnp.float32),
        mesh=mesh,
        scratch_types=[
            pltpu.VMEM((b_per_w,), jnp.int32),
            pltpu.VMEM((NBUF, CH, DP), jnp.float32),
            pltpu.SemaphoreType.DMA((NBUF,)),
            pltpu.SemaphoreType.DMA((NBUF,)),
        ],
    )
    def k(idx_hbm, table_hbm, out_hbm, idx_v, rows_v, gsem, ssem):
        wid = lax.axis_index("s") * 2 + lax.axis_index("c")
        base_b = wid * n_b
        pltpu.sync_copy(idx_hbm.at[pl.ds(wid * b_per_w, b_per_w)], idx_v)

        def gather(t, b):
            return pltpu.make_async_copy(
                table_hbm.at[idx_v.at[pl.ds(t * CH, CH)]], rows_v.at[b],
                gsem.at[b])

        def store(t, b):
            return pltpu.make_async_copy(
                rows_v.at[b], out_hbm.at[base_b + t], ssem.at[b])

        for b in range(NBUF):
            gather(b, b).start()

        def group(g, carry):
            for b in range(NBUF):
                t = g * NBUF + b
                gather(t, b).wait()
                store(t, b).start()

                @pl.when(g + 1 < n_grp)
                def _():
                    store(t, b).wait()
                    gather(t + NBUF, b).start()
            return carry

        lax.fori_loop(0, n_grp, group, 0)

        for b in range(NBUF):
            t = (n_grp - 1) * NBUF + b
            store(t, b).wait()

    return k(idx1d, tpad)


def kernel(inputs, emb_edges):
    V = emb_edges.shape[0]
    n_full = V // DP
    idx1d = inputs.reshape(-1)
    embT = jnp.transpose(emb_edges)
    tail = jnp.pad(
        emb_edges[n_full * DP:, :],
        ((0, DP - (V - n_full * DP)), (0, DP - D)))
    tpad = _convert(embT, tail)
    outp = _lookup(idx1d, tpad)
    return outp[:, :, :D]


# speedup vs baseline: 2.4530x; 2.4530x over previous
import functools

import jax
import jax.numpy as jnp
from jax import lax
from jax.experimental import pallas as pl
from jax.experimental.pallas import tpu as pltpu
from jax.experimental.pallas import tpu_sc as plsc

D = 64
DP = 128
NW = 32
NBUF = 4


@jax.jit
def _lookup(idx1d, tpad):
    B = idx1d.shape[0]
    b_per_w = B // NW
    n_b = 1024 // NW
    CH = 200
    n_grp = n_b // NBUF
    assert b_per_w == n_b * CH and n_b % NBUF == 0

    mesh = plsc.VectorSubcoreMesh(core_axis_name="c", subcore_axis_name="s")

    @functools.partial(
        pl.kernel,
        out_type=jax.ShapeDtypeStruct((1024, 200, DP), jnp.float32),
        mesh=mesh,
        scratch_types=[
            pltpu.VMEM((b_per_w,), jnp.int32),
            pltpu.VMEM((NBUF, CH, DP), jnp.float32),
            pltpu.SemaphoreType.DMA((NBUF,)),
            pltpu.SemaphoreType.DMA((NBUF,)),
        ],
    )
    def k(idx_hbm, table_hbm, out_hbm, idx_v, rows_v, gsem, ssem):
        wid = lax.axis_index("s") * 2 + lax.axis_index("c")
        base_b = wid * n_b
        pltpu.sync_copy(idx_hbm.at[pl.ds(wid * b_per_w, b_per_w)], idx_v)

        def gather(t, b):
            return pltpu.make_async_copy(
                table_hbm.at[idx_v.at[pl.ds(t * CH, CH)]], rows_v.at[b],
                gsem.at[b])

        def store(t, b):
            return pltpu.make_async_copy(
                rows_v.at[b], out_hbm.at[base_b + t], ssem.at[b])

        for b in range(NBUF):
            gather(b, b).start()

        def group(g, carry):
            for b in range(NBUF):
                t = g * NBUF + b
                gather(t, b).wait()
                store(t, b).start()

                @pl.when(g + 1 < n_grp)
                def _():
                    store(t, b).wait()
                    gather(t + NBUF, b).start()
            return carry

        lax.fori_loop(0, n_grp, group, 0)

        for b in range(NBUF):
            t = (n_grp - 1) * NBUF + b
            store(t, b).wait()

    return k(idx1d, tpad)


def kernel(inputs, emb_edges):
    idx1d = inputs.reshape(-1)
    tpad = jnp.pad(emb_edges, ((0, 0), (0, DP - D)))
    outp = _lookup(idx1d, tpad)
    return outp[:, :, :D]
